# Initial kernel scaffold; baseline (speedup 1.0000x reference)
import jax
import jax.numpy as jnp
from jax.experimental import pallas as pl


def kernel(x, edge_index, W_pre, b_pre, g_pre, be_pre, Wc1, bc1, Wc2, bc2, Wc3, bc3, W_post, b_post, g_post, be_post, W_f, b_f):
    raise NotImplementedError("write your pallas kernel here")


# speedup vs baseline: 6.8948x; 6.8948x over previous
import functools

import jax
import jax.numpy as jnp
from jax import lax
from jax.experimental import pallas as pl
from jax.experimental.pallas import tpu as pltpu
from jax.experimental.pallas import tpu_sc as plsc

NN = 10000
EE = 160000
DIN = 256
HH = 256
COUT = 64

NC = 2
NS = 16
LL = 16

HALF = HH // 2
E_PER_TILE = EE // NS
CHUNK = 128
NCHUNK = 80
E_PAD = NCHUNK * CHUNK
ACC_ROWS = 10112
ROWS_PER_TILE = ACC_ROWS // NS
ROWB = 10
BR = NN // ROWB
EPS = 1e-5

def _deg_body(dstp_hbm, zeros_hbm, out_hbm, dstv, ones_rows, acc):
    c = lax.axis_index("c")
    t = lax.axis_index("s")

    row0 = pl.multiple_of(t * ROWS_PER_TILE, 8)
    pltpu.sync_copy(dstp_hbm.at[t], dstv)
    pltpu.sync_copy(zeros_hbm.at[pl.ds(row0, ROWS_PER_TILE)],
                    acc.at[pl.ds(row0, ROWS_PER_TILE)])
    ones16 = jnp.ones((LL,), jnp.float32)

    def ones_body(j, _):
        for kk in range(HALF // LL):
            ones_rows[j, pl.ds(kk * LL, LL)] = ones16
        return 0

    lax.fori_loop(0, CHUNK, ones_body, 0)
    plsc.subcore_barrier()

    half_chunks = NCHUNK // 2

    @pl.when(c == 0)
    def _():
        for j in range(half_chunks):
            pltpu.sync_copy(ones_rows, acc.at[dstv.at[j]], add=True)

    @pl.when(c == 1)
    def _():
        for j in range(half_chunks, NCHUNK):
            pltpu.sync_copy(ones_rows, acc.at[dstv.at[j]], add=True)

    plsc.subcore_barrier()
    pltpu.sync_copy(acc.at[pl.ds(row0, ROWS_PER_TILE)],
                    out_hbm.at[c].at[pl.ds(row0, ROWS_PER_TILE)])


def _scatter_body(g_hbm, srcp_hbm, dstp_hbm, zeros_hbm, out_hbm,
                  srcv, dstv, rows, acc, sem):
    c = lax.axis_index("c")
    t = lax.axis_index("s")

    row0 = pl.multiple_of(t * ROWS_PER_TILE, 8)
    pltpu.sync_copy(srcp_hbm.at[t], srcv)
    pltpu.sync_copy(dstp_hbm.at[t], dstv)
    pltpu.sync_copy(zeros_hbm.at[pl.ds(row0, ROWS_PER_TILE)],
                    acc.at[pl.ds(row0, ROWS_PER_TILE)])
    plsc.subcore_barrier()

    table = g_hbm.at[c]
    for j in range(NCHUNK):
        pltpu.async_copy(table.at[srcv.at[j]], rows, sem).wait()
        pltpu.sync_copy(rows, acc.at[dstv.at[j]], add=True)

    plsc.subcore_barrier()

    last = NN - (NS - 1) * ROWS_PER_TILE

    @pl.when(t < NS - 1)
    def _():
        pltpu.sync_copy(acc.at[pl.ds(row0, ROWS_PER_TILE)],
                        out_hbm.at[c].at[pl.ds(row0, ROWS_PER_TILE)])

    @pl.when(t == NS - 1)
    def _():
        pltpu.sync_copy(acc.at[pl.ds((NS - 1) * ROWS_PER_TILE, last)],
                        out_hbm.at[c].at[pl.ds((NS - 1) * ROWS_PER_TILE, last)])


@functools.lru_cache(maxsize=1)
def _sc_kernels():
    mesh = plsc.VectorSubcoreMesh(
        core_axis_name="c", subcore_axis_name="s",
        num_cores=NC, num_subcores=NS)
    deg_kernel = pl.kernel(
        _deg_body,
        out_type=jax.ShapeDtypeStruct((NC, ACC_ROWS, HALF), jnp.float32),
        mesh=mesh,
        scratch_types=[
            pltpu.VMEM((NCHUNK, CHUNK), jnp.int32),
            pltpu.VMEM((CHUNK, HALF), jnp.float32),
            pltpu.VMEM_SHARED((ACC_ROWS, HALF), jnp.float32),
        ],
    )
    scatter_kernel = pl.kernel(
        _scatter_body,
        out_type=jax.ShapeDtypeStruct((NC, NN, HALF), jnp.float32),
        mesh=mesh,
        scratch_types=[
            pltpu.VMEM((NCHUNK, CHUNK), jnp.int32),
            pltpu.VMEM((NCHUNK, CHUNK), jnp.int32),
            pltpu.VMEM((CHUNK, HALF), jnp.float32),
            pltpu.VMEM_SHARED((ACC_ROWS, HALF), jnp.float32),
            pltpu.SemaphoreType.DMA,
        ],
    )
    return deg_kernel, scatter_kernel


def _k_pre(x_ref, w_ref, b_ref, deg_ref, y_ref, s_ref, q_ref, dinv_ref):
    r = pl.program_id(0)
    yb = jnp.dot(x_ref[...], w_ref[...], preferred_element_type=jnp.float32)
    yb = yb + b_ref[...]
    y_ref[...] = yb
    dinv_ref[...] = lax.rsqrt(deg_ref[...] + 1.0)

    @pl.when(r == 0)
    def _():
        s_ref[...] = jnp.zeros_like(s_ref)
        q_ref[...] = jnp.zeros_like(q_ref)

    s_ref[...] += jnp.sum(yb, axis=0, keepdims=True)
    q_ref[...] += jnp.sum(yb * yb, axis=0, keepdims=True)


def _bn_scale_shift(s_ref, q_ref, g_ref, be_ref):
    mu = s_ref[...] / NN
    var = q_ref[...] / NN - mu * mu
    sc = g_ref[...] * lax.rsqrt(var + EPS)
    sh = be_ref[...] - mu * sc
    return sc, sh


def _k_bn_mm(y_ref, s_ref, q_ref, g_ref, be_ref, dinv_ref, w_ref,
             h0_ref, gd_ref):
    sc, sh = _bn_scale_shift(s_ref, q_ref, g_ref, be_ref)
    h = jnp.maximum(y_ref[...] * sc + sh, 0.0)
    h0_ref[...] = h
    z = jnp.dot(dinv_ref[...] * h, w_ref[...], preferred_element_type=jnp.float32)
    gd_ref[0] = z[:, :HALF]
    gd_ref[1] = z[:, HALF:]


def _k_conv_mm(sa_ref, gd_ref, dinv_ref, bc_ref, h0_ref, w_ref, gdn_ref):
    u0 = sa_ref[0] + gd_ref[0]
    u1 = sa_ref[1] + gd_ref[1]
    u = jnp.concatenate([u0, u1], axis=1)
    h = dinv_ref[...] * u + bc_ref[...] + h0_ref[...]
    z = jnp.dot(dinv_ref[...] * h, w_ref[...], preferred_element_type=jnp.float32)
    gdn_ref[0] = z[:, :HALF]
    gdn_ref[1] = z[:, HALF:]


def _k_conv_post(sa_ref, gd_ref, dinv_ref, bc_ref, h0_ref, w_ref, b_ref,
                 wout_ref, s_ref, q_ref):
    r = pl.program_id(0)
    u0 = sa_ref[0] + gd_ref[0]
    u1 = sa_ref[1] + gd_ref[1]
    u = jnp.concatenate([u0, u1], axis=1)
    h = dinv_ref[...] * u + bc_ref[...] + h0_ref[...]
    wb = jnp.dot(h, w_ref[...], preferred_element_type=jnp.float32) + b_ref[...]
    wout_ref[...] = wb

    @pl.when(r == 0)
    def _():
        s_ref[...] = jnp.zeros_like(s_ref)
        q_ref[...] = jnp.zeros_like(q_ref)

    s_ref[...] += jnp.sum(wb, axis=0, keepdims=True)
    q_ref[...] += jnp.sum(wb * wb, axis=0, keepdims=True)


def _k_final(w_ref, s_ref, q_ref, g_ref, be_ref, wf_ref, bf_ref, out_ref):
    sc, sh = _bn_scale_shift(s_ref, q_ref, g_ref, be_ref)
    h = jnp.maximum(w_ref[...] * sc + sh, 0.0)
    tt = jnp.dot(h, wf_ref[...], preferred_element_type=jnp.float32) + bf_ref[...]
    m = jnp.max(tt, axis=1, keepdims=True)
    e = jnp.exp(tt - m)
    lse = jnp.log(jnp.sum(e, axis=1, keepdims=True))
    out_ref[...] = tt - m - lse


def _row_spec(cols):
    return pl.BlockSpec((BR, cols), lambda r: (r, 0))


def _full_spec(shape):
    return pl.BlockSpec(shape, lambda r: tuple(0 for _ in shape))


def _half_spec():
    return pl.BlockSpec((NC, BR, HALF), lambda r: (0, r, 0))


_VEC = _row_spec(1)
_STAT = _full_spec((1, HH))


def kernel(x, edge_index, W_pre, b_pre, g_pre, be_pre, Wc1, bc1, Wc2, bc2,
           Wc3, bc3, W_post, b_post, g_post, be_post, W_f, b_f):
    f32 = jnp.float32
    src = edge_index[0]
    dst = edge_index[1]

    pad_s = jnp.zeros((NS, E_PAD - E_PER_TILE), jnp.int32)
    pad_d = jnp.full((NS, E_PAD - E_PER_TILE), NN, jnp.int32)
    srcp = jnp.concatenate([src.reshape(NS, E_PER_TILE), pad_s], axis=1)
    srcp = srcp.reshape(NS, NCHUNK, CHUNK)
    dstp = jnp.concatenate([dst.reshape(NS, E_PER_TILE), pad_d], axis=1)
    dstp = dstp.reshape(NS, NCHUNK, CHUNK)
    zeros_acc = jnp.zeros((ACC_ROWS, HALF), f32)

    _deg_kernel, _scatter_kernel = _sc_kernels()
    hist = _deg_kernel(dstp, zeros_acc)
    deg = hist[0, :NN, 0:1] + hist[1, :NN, 0:1]

    b_pre2 = b_pre.reshape(1, HH)
    g_pre2 = g_pre.reshape(1, HH)
    be_pre2 = be_pre.reshape(1, HH)
    bc12 = bc1.reshape(1, HH)
    bc22 = bc2.reshape(1, HH)
    bc32 = bc3.reshape(1, HH)
    b_post2 = b_post.reshape(1, HH)
    g_post2 = g_post.reshape(1, HH)
    be_post2 = be_post.reshape(1, HH)
    b_f2 = b_f.reshape(1, COUT)

    y, s1, q1, dinv = pl.pallas_call(
        _k_pre,
        grid=(ROWB,),
        in_specs=[_row_spec(DIN), _full_spec((DIN, HH)), _STAT, _VEC],
        out_specs=[_row_spec(HH), _STAT, _STAT, _VEC],
        out_shape=[
            jax.ShapeDtypeStruct((NN, HH), f32),
            jax.ShapeDtypeStruct((1, HH), f32),
            jax.ShapeDtypeStruct((1, HH), f32),
            jax.ShapeDtypeStruct((NN, 1), f32),
        ],
    )(x, W_pre, b_pre2, deg)

    h0, gd1 = pl.pallas_call(
        _k_bn_mm,
        grid=(ROWB,),
        in_specs=[_row_spec(HH), _STAT, _STAT, _STAT, _STAT, _VEC,
                  _full_spec((HH, HH))],
        out_specs=[_row_spec(HH), _half_spec()],
        out_shape=[
            jax.ShapeDtypeStruct((NN, HH), f32),
            jax.ShapeDtypeStruct((NC, NN, HALF), f32),
        ],
    )(y, s1, q1, g_pre2, be_pre2, dinv, Wc1)

    sa1 = _scatter_kernel(gd1, srcp, dstp, zeros_acc)

    conv_call = pl.pallas_call(
        _k_conv_mm,
        grid=(ROWB,),
        in_specs=[_half_spec(), _half_spec(), _VEC, _STAT, _row_spec(HH),
                  _full_spec((HH, HH))],
        out_specs=_half_spec(),
        out_shape=jax.ShapeDtypeStruct((NC, NN, HALF), f32),
    )

    gd2 = conv_call(sa1, gd1, dinv, bc12, h0, Wc2)
    sa2 = _scatter_kernel(gd2, srcp, dstp, zeros_acc)

    gd3 = conv_call(sa2, gd2, dinv, bc22, h0, Wc3)
    sa3 = _scatter_kernel(gd3, srcp, dstp, zeros_acc)

    w, s2, q2 = pl.pallas_call(
        _k_conv_post,
        grid=(ROWB,),
        in_specs=[_half_spec(), _half_spec(), _VEC, _STAT, _row_spec(HH),
                  _full_spec((HH, HH)), _STAT],
        out_specs=[_row_spec(HH), _STAT, _STAT],
        out_shape=[
            jax.ShapeDtypeStruct((NN, HH), f32),
            jax.ShapeDtypeStruct((1, HH), f32),
            jax.ShapeDtypeStruct((1, HH), f32),
        ],
    )(sa3, gd3, dinv, bc32, h0, W_post, b_post2)

    out = pl.pallas_call(
        _k_final,
        grid=(ROWB,),
        in_specs=[_row_spec(HH), _STAT, _STAT, _STAT, _STAT,
                  _full_spec((HH, COUT)), _full_spec((1, COUT))],
        out_specs=_row_spec(COUT),
        out_shape=jax.ShapeDtypeStruct((NN, COUT), f32),
    )(w, s2, q2, g_post2, be_post2, W_f, b_f2)

    return out


# speedup vs baseline: 8.3139x; 1.2058x over previous
import functools

import jax
import jax.numpy as jnp
from jax import lax
from jax.experimental import pallas as pl
from jax.experimental.pallas import tpu as pltpu
from jax.experimental.pallas import tpu_sc as plsc

NN = 10000
EE = 160000
DIN = 256
HH = 256
COUT = 64

NC = 2
NS = 16
LL = 16

HALF = HH // 2
E_PER_TILE = EE // NS
CHUNK = 128
NCHUNK = 80
NBUF = 2
E_PAD = NCHUNK * CHUNK
ACC_ROWS = 10112
ROWS_PER_TILE = ACC_ROWS // NS
ROWB = 10
BR = NN // ROWB
EPS = 1e-5

def _deg_body(dstp_hbm, zeros_hbm, out_hbm, dstv, ones_rows, acc):
    c = lax.axis_index("c")
    t = lax.axis_index("s")

    row0 = pl.multiple_of(t * ROWS_PER_TILE, 8)
    pltpu.sync_copy(dstp_hbm.at[t], dstv)
    pltpu.sync_copy(zeros_hbm.at[pl.ds(row0, ROWS_PER_TILE)],
                    acc.at[pl.ds(row0, ROWS_PER_TILE)])
    ones16 = jnp.ones((LL,), jnp.float32)

    def ones_body(j, _):
        for kk in range(HALF // LL):
            ones_rows[j, pl.ds(kk * LL, LL)] = ones16
        return 0

    lax.fori_loop(0, CHUNK, ones_body, 0)
    plsc.subcore_barrier()

    half_chunks = NCHUNK // 2

    def scatter_ones(lo, hi):
        for j in range(lo, hi):
            pltpu.sync_copy(ones_rows, acc.at[dstv.at[j]], add=True)

    @pl.when(c == 0)
    def _():
        scatter_ones(0, half_chunks)

    @pl.when(c == 1)
    def _():
        scatter_ones(half_chunks, NCHUNK)

    plsc.subcore_barrier()
    pltpu.sync_copy(acc.at[pl.ds(row0, ROWS_PER_TILE)],
                    out_hbm.at[c].at[pl.ds(row0, ROWS_PER_TILE)])


def _scatter_body(g_hbm, srcp_hbm, dstp_hbm, zeros_hbm, out_hbm,
                  srcv, dstv, rows, acc, gs0, gs1):
    c = lax.axis_index("c")
    t = lax.axis_index("s")

    row0 = pl.multiple_of(t * ROWS_PER_TILE, 8)
    pltpu.sync_copy(dstp_hbm.at[t], dstv)
    pltpu.sync_copy(zeros_hbm.at[pl.ds(row0, ROWS_PER_TILE)],
                    acc.at[pl.ds(row0, ROWS_PER_TILE)])
    plsc.subcore_barrier()

    table = g_hbm.at[c]
    gsems = [gs0, gs1]
    half = NCHUNK // 2
    for h in range(2):
        base = h * half
        pltpu.sync_copy(srcp_hbm.at[t].at[pl.ds(base, half)], srcv)
        gds = {}
        for b in range(NBUF):
            gds[b] = pltpu.async_copy(table.at[srcv.at[b]], rows.at[b],
                                      gsems[b])
        for jj in range(half):
            j = base + jj
            b = jj % NBUF
            gds[b].wait()
            pltpu.sync_copy(rows.at[b], acc.at[dstv.at[j]], add=True)
            njj = jj + NBUF
            if njj < half:
                gds[b] = pltpu.async_copy(table.at[srcv.at[njj]], rows.at[b],
                                          gsems[b])

    plsc.subcore_barrier()

    last = NN - (NS - 1) * ROWS_PER_TILE

    @pl.when(t < NS - 1)
    def _():
        pltpu.sync_copy(acc.at[pl.ds(row0, ROWS_PER_TILE)],
                        out_hbm.at[c].at[pl.ds(row0, ROWS_PER_TILE)])

    @pl.when(t == NS - 1)
    def _():
        pltpu.sync_copy(acc.at[pl.ds((NS - 1) * ROWS_PER_TILE, last)],
                        out_hbm.at[c].at[pl.ds((NS - 1) * ROWS_PER_TILE, last)])


@functools.lru_cache(maxsize=1)
def _sc_kernels():
    mesh = plsc.VectorSubcoreMesh(
        core_axis_name="c", subcore_axis_name="s",
        num_cores=NC, num_subcores=NS)
    deg_kernel = pl.kernel(
        _deg_body,
        out_type=jax.ShapeDtypeStruct((NC, ACC_ROWS, HALF), jnp.float32),
        mesh=mesh,
        scratch_types=[
            pltpu.VMEM((NCHUNK, CHUNK), jnp.int32),
            pltpu.VMEM((CHUNK, HALF), jnp.float32),
            pltpu.VMEM_SHARED((ACC_ROWS, HALF), jnp.float32),
        ],
    )
    scatter_kernel = pl.kernel(
        _scatter_body,
        out_type=jax.ShapeDtypeStruct((NC, NN, HALF), jnp.float32),
        mesh=mesh,
        scratch_types=[
            pltpu.VMEM((NCHUNK // 2, CHUNK), jnp.int32),
            pltpu.VMEM((NCHUNK, CHUNK), jnp.int32),
            pltpu.VMEM((NBUF, CHUNK, HALF), jnp.float32),
            pltpu.VMEM_SHARED((ACC_ROWS, HALF), jnp.float32),
            pltpu.SemaphoreType.DMA,
            pltpu.SemaphoreType.DMA,
        ],
    )
    return deg_kernel, scatter_kernel


def _k_pre(x_ref, w_ref, b_ref, deg_ref, y_ref, s_ref, q_ref, dinv_ref):
    r = pl.program_id(0)
    yb = jnp.dot(x_ref[...], w_ref[...], preferred_element_type=jnp.float32)
    yb = yb + b_ref[...]
    y_ref[...] = yb
    dinv_ref[...] = lax.rsqrt(deg_ref[...] + 1.0)

    @pl.when(r == 0)
    def _():
        s_ref[...] = jnp.zeros_like(s_ref)
        q_ref[...] = jnp.zeros_like(q_ref)

    s_ref[...] += jnp.sum(yb, axis=0, keepdims=True)
    q_ref[...] += jnp.sum(yb * yb, axis=0, keepdims=True)


def _bn_scale_shift(s_ref, q_ref, g_ref, be_ref):
    mu = s_ref[...] / NN
    var = q_ref[...] / NN - mu * mu
    sc = g_ref[...] * lax.rsqrt(var + EPS)
    sh = be_ref[...] - mu * sc
    return sc, sh


def _k_bn_mm(y_ref, s_ref, q_ref, g_ref, be_ref, dinv_ref, w_ref,
             h0_ref, gd_ref):
    sc, sh = _bn_scale_shift(s_ref, q_ref, g_ref, be_ref)
    h = jnp.maximum(y_ref[...] * sc + sh, 0.0)
    h0_ref[...] = h
    z = jnp.dot(dinv_ref[...] * h, w_ref[...], preferred_element_type=jnp.float32)
    gd_ref[0] = z[:, :HALF]
    gd_ref[1] = z[:, HALF:]


def _k_conv_mm(sa_ref, gd_ref, dinv_ref, bc_ref, h0_ref, w_ref, gdn_ref):
    u0 = sa_ref[0] + gd_ref[0]
    u1 = sa_ref[1] + gd_ref[1]
    u = jnp.concatenate([u0, u1], axis=1)
    h = dinv_ref[...] * u + bc_ref[...] + h0_ref[...]
    z = jnp.dot(dinv_ref[...] * h, w_ref[...], preferred_element_type=jnp.float32)
    gdn_ref[0] = z[:, :HALF]
    gdn_ref[1] = z[:, HALF:]


def _k_conv_post(sa_ref, gd_ref, dinv_ref, bc_ref, h0_ref, w_ref, b_ref,
                 wout_ref, s_ref, q_ref):
    r = pl.program_id(0)
    u0 = sa_ref[0] + gd_ref[0]
    u1 = sa_ref[1] + gd_ref[1]
    u = jnp.concatenate([u0, u1], axis=1)
    h = dinv_ref[...] * u + bc_ref[...] + h0_ref[...]
    wb = jnp.dot(h, w_ref[...], preferred_element_type=jnp.float32) + b_ref[...]
    wout_ref[...] = wb

    @pl.when(r == 0)
    def _():
        s_ref[...] = jnp.zeros_like(s_ref)
        q_ref[...] = jnp.zeros_like(q_ref)

    s_ref[...] += jnp.sum(wb, axis=0, keepdims=True)
    q_ref[...] += jnp.sum(wb * wb, axis=0, keepdims=True)


def _k_final(w_ref, s_ref, q_ref, g_ref, be_ref, wf_ref, bf_ref, out_ref):
    sc, sh = _bn_scale_shift(s_ref, q_ref, g_ref, be_ref)
    h = jnp.maximum(w_ref[...] * sc + sh, 0.0)
    tt = jnp.dot(h, wf_ref[...], preferred_element_type=jnp.float32) + bf_ref[...]
    m = jnp.max(tt, axis=1, keepdims=True)
    e = jnp.exp(tt - m)
    lse = jnp.log(jnp.sum(e, axis=1, keepdims=True))
    out_ref[...] = tt - m - lse


def _row_spec(cols):
    return pl.BlockSpec((BR, cols), lambda r: (r, 0))


def _full_spec(shape):
    return pl.BlockSpec(shape, lambda r: tuple(0 for _ in shape))


def _half_spec():
    return pl.BlockSpec((NC, BR, HALF), lambda r: (0, r, 0))


_VEC = _row_spec(1)
_STAT = _full_spec((1, HH))


def kernel(x, edge_index, W_pre, b_pre, g_pre, be_pre, Wc1, bc1, Wc2, bc2,
           Wc3, bc3, W_post, b_post, g_post, be_post, W_f, b_f):
    f32 = jnp.float32
    src = edge_index[0]
    dst = edge_index[1]

    pad_s = jnp.zeros((NS, E_PAD - E_PER_TILE), jnp.int32)
    pad_d = jnp.full((NS, E_PAD - E_PER_TILE), NN, jnp.int32)
    srcp = jnp.concatenate([src.reshape(NS, E_PER_TILE), pad_s], axis=1)
    srcp = srcp.reshape(NS, NCHUNK, CHUNK)
    dstp = jnp.concatenate([dst.reshape(NS, E_PER_TILE), pad_d], axis=1)
    dstp = dstp.reshape(NS, NCHUNK, CHUNK)
    zeros_acc = jnp.zeros((ACC_ROWS, HALF), f32)

    _deg_kernel, _scatter_kernel = _sc_kernels()
    hist = _deg_kernel(dstp, zeros_acc)
    deg = hist[0, :NN, 0:1] + hist[1, :NN, 0:1]

    b_pre2 = b_pre.reshape(1, HH)
    g_pre2 = g_pre.reshape(1, HH)
    be_pre2 = be_pre.reshape(1, HH)
    bc12 = bc1.reshape(1, HH)
    bc22 = bc2.reshape(1, HH)
    bc32 = bc3.reshape(1, HH)
    b_post2 = b_post.reshape(1, HH)
    g_post2 = g_post.reshape(1, HH)
    be_post2 = be_post.reshape(1, HH)
    b_f2 = b_f.reshape(1, COUT)

    y, s1, q1, dinv = pl.pallas_call(
        _k_pre,
        grid=(ROWB,),
        in_specs=[_row_spec(DIN), _full_spec((DIN, HH)), _STAT, _VEC],
        out_specs=[_row_spec(HH), _STAT, _STAT, _VEC],
        out_shape=[
            jax.ShapeDtypeStruct((NN, HH), f32),
            jax.ShapeDtypeStruct((1, HH), f32),
            jax.ShapeDtypeStruct((1, HH), f32),
            jax.ShapeDtypeStruct((NN, 1), f32),
        ],
    )(x, W_pre, b_pre2, deg)

    h0, gd1 = pl.pallas_call(
        _k_bn_mm,
        grid=(ROWB,),
        in_specs=[_row_spec(HH), _STAT, _STAT, _STAT, _STAT, _VEC,
                  _full_spec((HH, HH))],
        out_specs=[_row_spec(HH), _half_spec()],
        out_shape=[
            jax.ShapeDtypeStruct((NN, HH), f32),
            jax.ShapeDtypeStruct((NC, NN, HALF), f32),
        ],
    )(y, s1, q1, g_pre2, be_pre2, dinv, Wc1)

    sa1 = _scatter_kernel(gd1, srcp, dstp, zeros_acc)

    conv_call = pl.pallas_call(
        _k_conv_mm,
        grid=(ROWB,),
        in_specs=[_half_spec(), _half_spec(), _VEC, _STAT, _row_spec(HH),
                  _full_spec((HH, HH))],
        out_specs=_half_spec(),
        out_shape=jax.ShapeDtypeStruct((NC, NN, HALF), f32),
    )

    gd2 = conv_call(sa1, gd1, dinv, bc12, h0, Wc2)
    sa2 = _scatter_kernel(gd2, srcp, dstp, zeros_acc)

    gd3 = conv_call(sa2, gd2, dinv, bc22, h0, Wc3)
    sa3 = _scatter_kernel(gd3, srcp, dstp, zeros_acc)

    w, s2, q2 = pl.pallas_call(
        _k_conv_post,
        grid=(ROWB,),
        in_specs=[_half_spec(), _half_spec(), _VEC, _STAT, _row_spec(HH),
                  _full_spec((HH, HH)), _STAT],
        out_specs=[_row_spec(HH), _STAT, _STAT],
        out_shape=[
            jax.ShapeDtypeStruct((NN, HH), f32),
            jax.ShapeDtypeStruct((1, HH), f32),
            jax.ShapeDtypeStruct((1, HH), f32),
        ],
    )(sa3, gd3, dinv, bc32, h0, W_post, b_post2)

    out = pl.pallas_call(
        _k_final,
        grid=(ROWB,),
        in_specs=[_row_spec(HH), _STAT, _STAT, _STAT, _STAT,
                  _full_spec((HH, COUT)), _full_spec((1, COUT))],
        out_specs=_row_spec(COUT),
        out_shape=jax.ShapeDtypeStruct((NN, COUT), f32),
    )(w, s2, q2, g_post2, be_post2, W_f, b_f2)

    return out


# speedup vs baseline: 8.3233x; 1.0011x over previous
import functools

import jax
import jax.numpy as jnp
from jax import lax
from jax.experimental import pallas as pl
from jax.experimental.pallas import tpu as pltpu
from jax.experimental.pallas import tpu_sc as plsc

NN = 10000
EE = 160000
DIN = 256
HH = 256
COUT = 64

NC = 2
NS = 16
LL = 16

HALF = HH // 2
E_PER_TILE = EE // NS
CHUNK = 128
NCHUNK = 80
NBUF = 2
E_PAD = NCHUNK * CHUNK
ACC_ROWS = 10112
ROWS_PER_TILE = ACC_ROWS // NS
ROWB = 10
BR = NN // ROWB
EPS = 1e-5

def _deg_body(dstp_hbm, zeros_hbm, out_hbm, dstv, ones_rows, acc, ds0, ds1):
    dsems = [ds0, ds1]
    c = lax.axis_index("c")
    t = lax.axis_index("s")

    row0 = pl.multiple_of(t * ROWS_PER_TILE, 8)
    pltpu.sync_copy(dstp_hbm.at[t], dstv)
    pltpu.sync_copy(zeros_hbm.at[pl.ds(row0, ROWS_PER_TILE)],
                    acc.at[pl.ds(row0, ROWS_PER_TILE)])
    ones16 = jnp.ones((LL,), jnp.float32)

    def ones_body(j, _):
        for kk in range(HALF // LL):
            ones_rows[j, pl.ds(kk * LL, LL)] = ones16
        return 0

    lax.fori_loop(0, CHUNK, ones_body, 0)
    plsc.subcore_barrier()

    half_chunks = NCHUNK // 2

    def scatter_ones(lo, hi):
        pend = {}
        for j in range(lo, hi):
            b = j % 2
            if b in pend:
                pend[b].wait()
            pend[b] = pltpu.async_copy(ones_rows, acc.at[dstv.at[j]],
                                       dsems[b], add=True)
        for b in sorted(pend):
            pend[b].wait()

    @pl.when(c == 0)
    def _():
        scatter_ones(0, half_chunks)

    @pl.when(c == 1)
    def _():
        scatter_ones(half_chunks, NCHUNK)

    plsc.subcore_barrier()
    pltpu.sync_copy(acc.at[pl.ds(row0, ROWS_PER_TILE)],
                    out_hbm.at[c].at[pl.ds(row0, ROWS_PER_TILE)])


def _scatter_body(g_hbm, srcp_hbm, dstp_hbm, zeros_hbm, out_hbm,
                  srcv, dstv, rows, acc, gs0, gs1, ss0, ss1):
    c = lax.axis_index("c")
    t = lax.axis_index("s")

    row0 = pl.multiple_of(t * ROWS_PER_TILE, 8)
    pltpu.sync_copy(dstp_hbm.at[t], dstv)
    pltpu.sync_copy(zeros_hbm.at[pl.ds(row0, ROWS_PER_TILE)],
                    acc.at[pl.ds(row0, ROWS_PER_TILE)])
    plsc.subcore_barrier()

    table = g_hbm.at[c]
    gsems = [gs0, gs1]
    ssems = [ss0, ss1]
    half = NCHUNK // 2
    for h in range(2):
        base = h * half
        pltpu.sync_copy(srcp_hbm.at[t].at[pl.ds(base, half)], srcv)
        gds = {}
        for b in range(NBUF):
            gds[b] = pltpu.async_copy(table.at[srcv.at[b]], rows.at[b],
                                      gsems[b])
        sds = {}
        for jj in range(half):
            j = base + jj
            b = jj % NBUF
            gds[b].wait()
            sds[b] = pltpu.async_copy(rows.at[b], acc.at[dstv.at[j]],
                                      ssems[b], add=True)
            njj = jj + NBUF
            if njj < half:
                ob = njj % NBUF
                sds.pop(ob).wait()
                gds[ob] = pltpu.async_copy(table.at[srcv.at[njj]],
                                           rows.at[ob], gsems[ob])
        for b in sorted(sds):
            sds[b].wait()

    plsc.subcore_barrier()

    last = NN - (NS - 1) * ROWS_PER_TILE

    @pl.when(t < NS - 1)
    def _():
        pltpu.sync_copy(acc.at[pl.ds(row0, ROWS_PER_TILE)],
                        out_hbm.at[c].at[pl.ds(row0, ROWS_PER_TILE)])

    @pl.when(t == NS - 1)
    def _():
        pltpu.sync_copy(acc.at[pl.ds((NS - 1) * ROWS_PER_TILE, last)],
                        out_hbm.at[c].at[pl.ds((NS - 1) * ROWS_PER_TILE, last)])


@functools.lru_cache(maxsize=1)
def _sc_kernels():
    mesh = plsc.VectorSubcoreMesh(
        core_axis_name="c", subcore_axis_name="s",
        num_cores=NC, num_subcores=NS)
    deg_kernel = pl.kernel(
        _deg_body,
        out_type=jax.ShapeDtypeStruct((NC, ACC_ROWS, HALF), jnp.float32),
        mesh=mesh,
        scratch_types=[
            pltpu.VMEM((NCHUNK, CHUNK), jnp.int32),
            pltpu.VMEM((CHUNK, HALF), jnp.float32),
            pltpu.VMEM_SHARED((ACC_ROWS, HALF), jnp.float32),
            pltpu.SemaphoreType.DMA,
            pltpu.SemaphoreType.DMA,
        ],
    )
    scatter_kernel = pl.kernel(
        _scatter_body,
        out_type=jax.ShapeDtypeStruct((NC, NN, HALF), jnp.float32),
        mesh=mesh,
        scratch_types=[
            pltpu.VMEM((NCHUNK // 2, CHUNK), jnp.int32),
            pltpu.VMEM((NCHUNK, CHUNK), jnp.int32),
            pltpu.VMEM((NBUF, CHUNK, HALF), jnp.float32),
            pltpu.VMEM_SHARED((ACC_ROWS, HALF), jnp.float32),
            pltpu.SemaphoreType.DMA,
            pltpu.SemaphoreType.DMA,
            pltpu.SemaphoreType.DMA,
            pltpu.SemaphoreType.DMA,
        ],
    )
    return deg_kernel, scatter_kernel


def _k_pre(x_ref, w_ref, b_ref, deg_ref, y_ref, s_ref, q_ref, dinv_ref):
    r = pl.program_id(0)
    yb = jnp.dot(x_ref[...], w_ref[...], preferred_element_type=jnp.float32)
    yb = yb + b_ref[...]
    y_ref[...] = yb
    dinv_ref[...] = lax.rsqrt(deg_ref[...] + 1.0)

    @pl.when(r == 0)
    def _():
        s_ref[...] = jnp.zeros_like(s_ref)
        q_ref[...] = jnp.zeros_like(q_ref)

    s_ref[...] += jnp.sum(yb, axis=0, keepdims=True)
    q_ref[...] += jnp.sum(yb * yb, axis=0, keepdims=True)


def _bn_scale_shift(s_ref, q_ref, g_ref, be_ref):
    mu = s_ref[...] / NN
    var = q_ref[...] / NN - mu * mu
    sc = g_ref[...] * lax.rsqrt(var + EPS)
    sh = be_ref[...] - mu * sc
    return sc, sh


def _k_bn_mm(y_ref, s_ref, q_ref, g_ref, be_ref, dinv_ref, w_ref,
             h0_ref, gd_ref):
    sc, sh = _bn_scale_shift(s_ref, q_ref, g_ref, be_ref)
    h = jnp.maximum(y_ref[...] * sc + sh, 0.0)
    h0_ref[...] = h
    z = jnp.dot(dinv_ref[...] * h, w_ref[...], preferred_element_type=jnp.float32)
    gd_ref[0] = z[:, :HALF]
    gd_ref[1] = z[:, HALF:]


def _k_conv_mm(sa_ref, gd_ref, dinv_ref, bc_ref, h0_ref, w_ref, gdn_ref):
    u0 = sa_ref[0] + gd_ref[0]
    u1 = sa_ref[1] + gd_ref[1]
    u = jnp.concatenate([u0, u1], axis=1)
    h = dinv_ref[...] * u + bc_ref[...] + h0_ref[...]
    z = jnp.dot(dinv_ref[...] * h, w_ref[...], preferred_element_type=jnp.float32)
    gdn_ref[0] = z[:, :HALF]
    gdn_ref[1] = z[:, HALF:]


def _k_conv_post(sa_ref, gd_ref, dinv_ref, bc_ref, h0_ref, w_ref, b_ref,
                 wout_ref, s_ref, q_ref):
    r = pl.program_id(0)
    u0 = sa_ref[0] + gd_ref[0]
    u1 = sa_ref[1] + gd_ref[1]
    u = jnp.concatenate([u0, u1], axis=1)
    h = dinv_ref[...] * u + bc_ref[...] + h0_ref[...]
    wb = jnp.dot(h, w_ref[...], preferred_element_type=jnp.float32) + b_ref[...]
    wout_ref[...] = wb

    @pl.when(r == 0)
    def _():
        s_ref[...] = jnp.zeros_like(s_ref)
        q_ref[...] = jnp.zeros_like(q_ref)

    s_ref[...] += jnp.sum(wb, axis=0, keepdims=True)
    q_ref[...] += jnp.sum(wb * wb, axis=0, keepdims=True)


def _k_final(w_ref, s_ref, q_ref, g_ref, be_ref, wf_ref, bf_ref, out_ref):
    sc, sh = _bn_scale_shift(s_ref, q_ref, g_ref, be_ref)
    h = jnp.maximum(w_ref[...] * sc + sh, 0.0)
    tt = jnp.dot(h, wf_ref[...], preferred_element_type=jnp.float32) + bf_ref[...]
    m = jnp.max(tt, axis=1, keepdims=True)
    e = jnp.exp(tt - m)
    lse = jnp.log(jnp.sum(e, axis=1, keepdims=True))
    out_ref[...] = tt - m - lse


def _row_spec(cols):
    return pl.BlockSpec((BR, cols), lambda r: (r, 0))


def _full_spec(shape):
    return pl.BlockSpec(shape, lambda r: tuple(0 for _ in shape))


def _half_spec():
    return pl.BlockSpec((NC, BR, HALF), lambda r: (0, r, 0))


_VEC = _row_spec(1)
_STAT = _full_spec((1, HH))


def kernel(x, edge_index, W_pre, b_pre, g_pre, be_pre, Wc1, bc1, Wc2, bc2,
           Wc3, bc3, W_post, b_post, g_post, be_post, W_f, b_f):
    f32 = jnp.float32
    src = edge_index[0]
    dst = edge_index[1]

    pad_s = jnp.zeros((NS, E_PAD - E_PER_TILE), jnp.int32)
    pad_d = jnp.full((NS, E_PAD - E_PER_TILE), NN, jnp.int32)
    srcp = jnp.concatenate([src.reshape(NS, E_PER_TILE), pad_s], axis=1)
    srcp = srcp.reshape(NS, NCHUNK, CHUNK)
    dstp = jnp.concatenate([dst.reshape(NS, E_PER_TILE), pad_d], axis=1)
    dstp = dstp.reshape(NS, NCHUNK, CHUNK)
    zeros_acc = jnp.zeros((ACC_ROWS, HALF), f32)

    _deg_kernel, _scatter_kernel = _sc_kernels()
    hist = _deg_kernel(dstp, zeros_acc)
    deg = hist[0, :NN, 0:1] + hist[1, :NN, 0:1]

    b_pre2 = b_pre.reshape(1, HH)
    g_pre2 = g_pre.reshape(1, HH)
    be_pre2 = be_pre.reshape(1, HH)
    bc12 = bc1.reshape(1, HH)
    bc22 = bc2.reshape(1, HH)
    bc32 = bc3.reshape(1, HH)
    b_post2 = b_post.reshape(1, HH)
    g_post2 = g_post.reshape(1, HH)
    be_post2 = be_post.reshape(1, HH)
    b_f2 = b_f.reshape(1, COUT)

    y, s1, q1, dinv = pl.pallas_call(
        _k_pre,
        grid=(ROWB,),
        in_specs=[_row_spec(DIN), _full_spec((DIN, HH)), _STAT, _VEC],
        out_specs=[_row_spec(HH), _STAT, _STAT, _VEC],
        out_shape=[
            jax.ShapeDtypeStruct((NN, HH), f32),
            jax.ShapeDtypeStruct((1, HH), f32),
            jax.ShapeDtypeStruct((1, HH), f32),
            jax.ShapeDtypeStruct((NN, 1), f32),
        ],
    )(x, W_pre, b_pre2, deg)

    h0, gd1 = pl.pallas_call(
        _k_bn_mm,
        grid=(ROWB,),
        in_specs=[_row_spec(HH), _STAT, _STAT, _STAT, _STAT, _VEC,
                  _full_spec((HH, HH))],
        out_specs=[_row_spec(HH), _half_spec()],
        out_shape=[
            jax.ShapeDtypeStruct((NN, HH), f32),
            jax.ShapeDtypeStruct((NC, NN, HALF), f32),
        ],
    )(y, s1, q1, g_pre2, be_pre2, dinv, Wc1)

    sa1 = _scatter_kernel(gd1, srcp, dstp, zeros_acc)

    conv_call = pl.pallas_call(
        _k_conv_mm,
        grid=(ROWB,),
        in_specs=[_half_spec(), _half_spec(), _VEC, _STAT, _row_spec(HH),
                  _full_spec((HH, HH))],
        out_specs=_half_spec(),
        out_shape=jax.ShapeDtypeStruct((NC, NN, HALF), f32),
    )

    gd2 = conv_call(sa1, gd1, dinv, bc12, h0, Wc2)
    sa2 = _scatter_kernel(gd2, srcp, dstp, zeros_acc)

    gd3 = conv_call(sa2, gd2, dinv, bc22, h0, Wc3)
    sa3 = _scatter_kernel(gd3, srcp, dstp, zeros_acc)

    w, s2, q2 = pl.pallas_call(
        _k_conv_post,
        grid=(ROWB,),
        in_specs=[_half_spec(), _half_spec(), _VEC, _STAT, _row_spec(HH),
                  _full_spec((HH, HH)), _STAT],
        out_specs=[_row_spec(HH), _STAT, _STAT],
        out_shape=[
            jax.ShapeDtypeStruct((NN, HH), f32),
            jax.ShapeDtypeStruct((1, HH), f32),
            jax.ShapeDtypeStruct((1, HH), f32),
        ],
    )(sa3, gd3, dinv, bc32, h0, W_post, b_post2)

    out = pl.pallas_call(
        _k_final,
        grid=(ROWB,),
        in_specs=[_row_spec(HH), _STAT, _STAT, _STAT, _STAT,
                  _full_spec((HH, COUT)), _full_spec((1, COUT))],
        out_specs=_row_spec(COUT),
        out_shape=jax.ShapeDtypeStruct((NN, COUT), f32),
    )(w, s2, q2, g_post2, be_post2, W_f, b_f2)

    return out


# speedup vs baseline: 8.6105x; 1.0345x over previous
import functools

import jax
import jax.numpy as jnp
from jax import lax
from jax.experimental import pallas as pl
from jax.experimental.pallas import tpu as pltpu
from jax.experimental.pallas import tpu_sc as plsc

NN = 10000
EE = 160000
DIN = 256
HH = 256
COUT = 64

NC = 2
NS = 16
LL = 16

HALF = HH // 2
E_PER_TILE = EE // NS
CHUNK = 128
NCHUNK = 80
NBUF = 2
E_PAD = NCHUNK * CHUNK
ACC_ROWS = 10112
ROWS_PER_TILE = ACC_ROWS // NS
ROWB = 10
BR = NN // ROWB
EPS = 1e-5

def _zero_acc_slice(zbuf, acc, row0):
    nfull = ROWS_PER_TILE // CHUNK
    rem = ROWS_PER_TILE - nfull * CHUNK
    for k in range(nfull):
        pltpu.sync_copy(zbuf.at[pl.ds(0, CHUNK)],
                        acc.at[pl.ds(row0 + k * CHUNK, CHUNK)])
    if rem:
        pltpu.sync_copy(zbuf.at[pl.ds(0, rem)],
                        acc.at[pl.ds(row0 + nfull * CHUNK, rem)])


def _deg_body(dstp_hbm, out_hbm, dstv, ones_rows, acc, ds0, ds1):
    dsems = [ds0, ds1]
    c = lax.axis_index("c")
    t = lax.axis_index("s")

    row0 = pl.multiple_of(t * ROWS_PER_TILE, 8)
    pltpu.sync_copy(dstp_hbm.at[t], dstv)
    zeros16 = jnp.zeros((LL,), jnp.float32)

    def zero_body(j, _):
        for kk in range(HALF // LL):
            ones_rows[j, pl.ds(kk * LL, LL)] = zeros16
        return 0

    lax.fori_loop(0, CHUNK, zero_body, 0)
    _zero_acc_slice(ones_rows, acc, row0)
    ones16 = jnp.ones((LL,), jnp.float32)

    def ones_body(j, _):
        for kk in range(HALF // LL):
            ones_rows[j, pl.ds(kk * LL, LL)] = ones16
        return 0

    lax.fori_loop(0, CHUNK, ones_body, 0)
    plsc.subcore_barrier()

    half_chunks = NCHUNK // 2

    def scatter_ones(lo, hi):
        pend = {}
        for j in range(lo, hi):
            b = j % 2
            if b in pend:
                pend[b].wait()
            pend[b] = pltpu.async_copy(ones_rows, acc.at[dstv.at[j]],
                                       dsems[b], add=True)
        for b in sorted(pend):
            pend[b].wait()

    @pl.when(c == 0)
    def _():
        scatter_ones(0, half_chunks)

    @pl.when(c == 1)
    def _():
        scatter_ones(half_chunks, NCHUNK)

    plsc.subcore_barrier()
    pltpu.sync_copy(acc.at[pl.ds(row0, ROWS_PER_TILE)],
                    out_hbm.at[c].at[pl.ds(row0, ROWS_PER_TILE)])


def _scatter_body(g_hbm, srcp_hbm, dstp_hbm, out_hbm,
                  srcv, dstv, rows, acc, gs0, gs1, ss0, ss1):
    c = lax.axis_index("c")
    t = lax.axis_index("s")

    row0 = pl.multiple_of(t * ROWS_PER_TILE, 8)
    pltpu.sync_copy(dstp_hbm.at[t], dstv)
    zeros16 = jnp.zeros((LL,), jnp.float32)

    def zero_body(j, _):
        for kk in range(HALF // LL):
            rows[0, j, pl.ds(kk * LL, LL)] = zeros16
        return 0

    lax.fori_loop(0, CHUNK, zero_body, 0)
    _zero_acc_slice(rows.at[0], acc, row0)
    plsc.subcore_barrier()

    table = g_hbm.at[c]
    gsems = [gs0, gs1]
    ssems = [ss0, ss1]
    half = NCHUNK // 2
    for h in range(2):
        base = h * half
        pltpu.sync_copy(srcp_hbm.at[t].at[pl.ds(base, half)], srcv)
        gds = {}
        for b in range(NBUF):
            gds[b] = pltpu.async_copy(table.at[srcv.at[b]], rows.at[b],
                                      gsems[b])
        sds = {}
        for jj in range(half):
            j = base + jj
            b = jj % NBUF
            gds[b].wait()
            sds[b] = pltpu.async_copy(rows.at[b], acc.at[dstv.at[j]],
                                      ssems[b], add=True)
            njj = jj + NBUF
            if njj < half:
                ob = njj % NBUF
                sds.pop(ob).wait()
                gds[ob] = pltpu.async_copy(table.at[srcv.at[njj]],
                                           rows.at[ob], gsems[ob])
        for b in sorted(sds):
            sds[b].wait()

    plsc.subcore_barrier()

    last = NN - (NS - 1) * ROWS_PER_TILE

    @pl.when(t < NS - 1)
    def _():
        pltpu.sync_copy(acc.at[pl.ds(row0, ROWS_PER_TILE)],
                        out_hbm.at[c].at[pl.ds(row0, ROWS_PER_TILE)])

    @pl.when(t == NS - 1)
    def _():
        pltpu.sync_copy(acc.at[pl.ds((NS - 1) * ROWS_PER_TILE, last)],
                        out_hbm.at[c].at[pl.ds((NS - 1) * ROWS_PER_TILE, last)])


@functools.lru_cache(maxsize=1)
def _sc_kernels():
    mesh = plsc.VectorSubcoreMesh(
        core_axis_name="c", subcore_axis_name="s",
        num_cores=NC, num_subcores=NS)
    deg_kernel = pl.kernel(
        _deg_body,
        out_type=jax.ShapeDtypeStruct((NC, ACC_ROWS, HALF), jnp.float32),
        mesh=mesh,
        scratch_types=[
            pltpu.VMEM((NCHUNK, CHUNK), jnp.int32),
            pltpu.VMEM((CHUNK, HALF), jnp.float32),
            pltpu.VMEM_SHARED((ACC_ROWS, HALF), jnp.float32),
            pltpu.SemaphoreType.DMA,
            pltpu.SemaphoreType.DMA,
        ],
    )
    scatter_kernel = pl.kernel(
        _scatter_body,
        out_type=jax.ShapeDtypeStruct((NC, NN, HALF), jnp.float32),
        mesh=mesh,
        scratch_types=[
            pltpu.VMEM((NCHUNK // 2, CHUNK), jnp.int32),
            pltpu.VMEM((NCHUNK, CHUNK), jnp.int32),
            pltpu.VMEM((NBUF, CHUNK, HALF), jnp.float32),
            pltpu.VMEM_SHARED((ACC_ROWS, HALF), jnp.float32),
            pltpu.SemaphoreType.DMA,
            pltpu.SemaphoreType.DMA,
            pltpu.SemaphoreType.DMA,
            pltpu.SemaphoreType.DMA,
        ],
    )
    return deg_kernel, scatter_kernel


def _k_pre(x_ref, w_ref, b_ref, y_ref, s_ref, q_ref):
    r = pl.program_id(0)
    yb = jnp.dot(x_ref[...], w_ref[...], preferred_element_type=jnp.float32)
    yb = yb + b_ref[...]
    y_ref[...] = yb

    @pl.when(r == 0)
    def _():
        s_ref[...] = jnp.zeros_like(s_ref)
        q_ref[...] = jnp.zeros_like(q_ref)

    s_ref[...] += jnp.sum(yb, axis=0, keepdims=True)
    q_ref[...] += jnp.sum(yb * yb, axis=0, keepdims=True)


def _bn_scale_shift(s_ref, q_ref, g_ref, be_ref):
    mu = s_ref[...] / NN
    var = q_ref[...] / NN - mu * mu
    sc = g_ref[...] * lax.rsqrt(var + EPS)
    sh = be_ref[...] - mu * sc
    return sc, sh


def _k_bn_mm(y_ref, s_ref, q_ref, g_ref, be_ref, deg_ref, w_ref,
             h0_ref, gd_ref, dinv_ref):
    sc, sh = _bn_scale_shift(s_ref, q_ref, g_ref, be_ref)
    dinv = lax.rsqrt(deg_ref[...] + 1.0)
    dinv_ref[...] = dinv
    h = jnp.maximum(y_ref[...] * sc + sh, 0.0)
    h0_ref[...] = h
    z = jnp.dot(dinv * h, w_ref[...], preferred_element_type=jnp.float32)
    gd_ref[0] = z[:, :HALF]
    gd_ref[1] = z[:, HALF:]


def _k_conv_mm(sa_ref, gd_ref, dinv_ref, bc_ref, h0_ref, w_ref, gdn_ref):
    u0 = sa_ref[0] + gd_ref[0]
    u1 = sa_ref[1] + gd_ref[1]
    u = jnp.concatenate([u0, u1], axis=1)
    h = dinv_ref[...] * u + bc_ref[...] + h0_ref[...]
    z = jnp.dot(dinv_ref[...] * h, w_ref[...], preferred_element_type=jnp.float32)
    gdn_ref[0] = z[:, :HALF]
    gdn_ref[1] = z[:, HALF:]


def _k_conv_post(sa_ref, gd_ref, dinv_ref, bc_ref, h0_ref, w_ref, b_ref,
                 wout_ref, s_ref, q_ref):
    r = pl.program_id(0)
    u0 = sa_ref[0] + gd_ref[0]
    u1 = sa_ref[1] + gd_ref[1]
    u = jnp.concatenate([u0, u1], axis=1)
    h = dinv_ref[...] * u + bc_ref[...] + h0_ref[...]
    wb = jnp.dot(h, w_ref[...], preferred_element_type=jnp.float32) + b_ref[...]
    wout_ref[...] = wb

    @pl.when(r == 0)
    def _():
        s_ref[...] = jnp.zeros_like(s_ref)
        q_ref[...] = jnp.zeros_like(q_ref)

    s_ref[...] += jnp.sum(wb, axis=0, keepdims=True)
    q_ref[...] += jnp.sum(wb * wb, axis=0, keepdims=True)


def _k_final(w_ref, s_ref, q_ref, g_ref, be_ref, wf_ref, bf_ref, out_ref):
    sc, sh = _bn_scale_shift(s_ref, q_ref, g_ref, be_ref)
    h = jnp.maximum(w_ref[...] * sc + sh, 0.0)
    tt = jnp.dot(h, wf_ref[...], preferred_element_type=jnp.float32) + bf_ref[...]
    m = jnp.max(tt, axis=1, keepdims=True)
    e = jnp.exp(tt - m)
    lse = jnp.log(jnp.sum(e, axis=1, keepdims=True))
    out_ref[...] = tt - m - lse


def _row_spec(cols):
    return pl.BlockSpec((BR, cols), lambda r: (r, 0))


def _full_spec(shape):
    return pl.BlockSpec(shape, lambda r: tuple(0 for _ in shape))


def _half_spec():
    return pl.BlockSpec((NC, BR, HALF), lambda r: (0, r, 0))


_VEC = _row_spec(1)
_STAT = _full_spec((1, HH))


def kernel(x, edge_index, W_pre, b_pre, g_pre, be_pre, Wc1, bc1, Wc2, bc2,
           Wc3, bc3, W_post, b_post, g_post, be_post, W_f, b_f):
    f32 = jnp.float32
    src = edge_index[0]
    dst = edge_index[1]

    pad_s = jnp.zeros((NS, E_PAD - E_PER_TILE), jnp.int32)
    pad_d = jnp.full((NS, E_PAD - E_PER_TILE), NN, jnp.int32)
    srcp = jnp.concatenate([src.reshape(NS, E_PER_TILE), pad_s], axis=1)
    srcp = srcp.reshape(NS, NCHUNK, CHUNK)
    dstp = jnp.concatenate([dst.reshape(NS, E_PER_TILE), pad_d], axis=1)
    dstp = dstp.reshape(NS, NCHUNK, CHUNK)
    _deg_kernel, _scatter_kernel = _sc_kernels()
    hist = _deg_kernel(dstp)
    deg = hist[0, :NN, 0:1] + hist[1, :NN, 0:1]

    b_pre2 = b_pre.reshape(1, HH)
    g_pre2 = g_pre.reshape(1, HH)
    be_pre2 = be_pre.reshape(1, HH)
    bc12 = bc1.reshape(1, HH)
    bc22 = bc2.reshape(1, HH)
    bc32 = bc3.reshape(1, HH)
    b_post2 = b_post.reshape(1, HH)
    g_post2 = g_post.reshape(1, HH)
    be_post2 = be_post.reshape(1, HH)
    b_f2 = b_f.reshape(1, COUT)

    y, s1, q1 = pl.pallas_call(
        _k_pre,
        grid=(ROWB,),
        in_specs=[_row_spec(DIN), _full_spec((DIN, HH)), _STAT],
        out_specs=[_row_spec(HH), _STAT, _STAT],
        out_shape=[
            jax.ShapeDtypeStruct((NN, HH), f32),
            jax.ShapeDtypeStruct((1, HH), f32),
            jax.ShapeDtypeStruct((1, HH), f32),
        ],
    )(x, W_pre, b_pre2)

    h0, gd1, dinv = pl.pallas_call(
        _k_bn_mm,
        grid=(ROWB,),
        in_specs=[_row_spec(HH), _STAT, _STAT, _STAT, _STAT, _VEC,
                  _full_spec((HH, HH))],
        out_specs=[_row_spec(HH), _half_spec(), _VEC],
        out_shape=[
            jax.ShapeDtypeStruct((NN, HH), f32),
            jax.ShapeDtypeStruct((NC, NN, HALF), f32),
            jax.ShapeDtypeStruct((NN, 1), f32),
        ],
    )(y, s1, q1, g_pre2, be_pre2, deg, Wc1)

    sa1 = _scatter_kernel(gd1, srcp, dstp)

    conv_call = pl.pallas_call(
        _k_conv_mm,
        grid=(ROWB,),
        in_specs=[_half_spec(), _half_spec(), _VEC, _STAT, _row_spec(HH),
                  _full_spec((HH, HH))],
        out_specs=_half_spec(),
        out_shape=jax.ShapeDtypeStruct((NC, NN, HALF), f32),
    )

    gd2 = conv_call(sa1, gd1, dinv, bc12, h0, Wc2)
    sa2 = _scatter_kernel(gd2, srcp, dstp)

    gd3 = conv_call(sa2, gd2, dinv, bc22, h0, Wc3)
    sa3 = _scatter_kernel(gd3, srcp, dstp)

    w, s2, q2 = pl.pallas_call(
        _k_conv_post,
        grid=(ROWB,),
        in_specs=[_half_spec(), _half_spec(), _VEC, _STAT, _row_spec(HH),
                  _full_spec((HH, HH)), _STAT],
        out_specs=[_row_spec(HH), _STAT, _STAT],
        out_shape=[
            jax.ShapeDtypeStruct((NN, HH), f32),
            jax.ShapeDtypeStruct((1, HH), f32),
            jax.ShapeDtypeStruct((1, HH), f32),
        ],
    )(sa3, gd3, dinv, bc32, h0, W_post, b_post2)

    out = pl.pallas_call(
        _k_final,
        grid=(ROWB,),
        in_specs=[_row_spec(HH), _STAT, _STAT, _STAT, _STAT,
                  _full_spec((HH, COUT)), _full_spec((1, COUT))],
        out_specs=_row_spec(COUT),
        out_shape=jax.ShapeDtypeStruct((NN, COUT), f32),
    )(w, s2, q2, g_post2, be_post2, W_f, b_f2)

    return out


# speedup vs baseline: 8.6843x; 1.0086x over previous
import functools

import jax
import jax.numpy as jnp
from jax import lax
from jax.experimental import pallas as pl
from jax.experimental.pallas import tpu as pltpu
from jax.experimental.pallas import tpu_sc as plsc

NN = 10000
EE = 160000
DIN = 256
HH = 256
COUT = 64

NC = 2
NS = 16
LL = 16

HALF = HH // 2
E_PER_TILE = EE // NS
CHUNK = 128
NCHUNK = 80
NBUF = 2
E_PAD = NCHUNK * CHUNK
ACC_ROWS = 10112
ROWS_PER_TILE = ACC_ROWS // NS
ROWB = 5
BR = NN // ROWB
EPS = 1e-5

def _zero_acc_slice(zbuf, acc, row0):
    nfull = ROWS_PER_TILE // CHUNK
    rem = ROWS_PER_TILE - nfull * CHUNK
    for k in range(nfull):
        pltpu.sync_copy(zbuf.at[pl.ds(0, CHUNK)],
                        acc.at[pl.ds(row0 + k * CHUNK, CHUNK)])
    if rem:
        pltpu.sync_copy(zbuf.at[pl.ds(0, rem)],
                        acc.at[pl.ds(row0 + nfull * CHUNK, rem)])


def _deg_body(dstp_hbm, out_hbm, dstv, ones_rows, acc, ds0, ds1):
    dsems = [ds0, ds1]
    c = lax.axis_index("c")
    t = lax.axis_index("s")

    row0 = pl.multiple_of(t * ROWS_PER_TILE, 8)
    pltpu.sync_copy(dstp_hbm.at[t], dstv)
    zeros16 = jnp.zeros((LL,), jnp.float32)

    def zero_body(j, _):
        for kk in range(HALF // LL):
            ones_rows[j, pl.ds(kk * LL, LL)] = zeros16
        return 0

    lax.fori_loop(0, CHUNK, zero_body, 0)
    _zero_acc_slice(ones_rows, acc, row0)
    ones16 = jnp.ones((LL,), jnp.float32)

    def ones_body(j, _):
        for kk in range(HALF // LL):
            ones_rows[j, pl.ds(kk * LL, LL)] = ones16
        return 0

    lax.fori_loop(0, CHUNK, ones_body, 0)
    plsc.subcore_barrier()

    half_chunks = NCHUNK // 2

    def scatter_ones(lo, hi):
        pend = {}
        for j in range(lo, hi):
            b = j % 2
            if b in pend:
                pend[b].wait()
            pend[b] = pltpu.async_copy(ones_rows, acc.at[dstv.at[j]],
                                       dsems[b], add=True)
        for b in sorted(pend):
            pend[b].wait()

    @pl.when(c == 0)
    def _():
        scatter_ones(0, half_chunks)

    @pl.when(c == 1)
    def _():
        scatter_ones(half_chunks, NCHUNK)

    plsc.subcore_barrier()
    pltpu.sync_copy(acc.at[pl.ds(row0, ROWS_PER_TILE)],
                    out_hbm.at[c].at[pl.ds(row0, ROWS_PER_TILE)])


def _scatter_body(g_hbm, srcp_hbm, dstp_hbm, out_hbm,
                  srcv, dstv, rows, acc, gs0, gs1, ss0, ss1):
    c = lax.axis_index("c")
    t = lax.axis_index("s")

    row0 = pl.multiple_of(t * ROWS_PER_TILE, 8)
    pltpu.sync_copy(dstp_hbm.at[t], dstv)
    zeros16 = jnp.zeros((LL,), jnp.float32)

    def zero_body(j, _):
        for kk in range(HALF // LL):
            rows[0, j, pl.ds(kk * LL, LL)] = zeros16
        return 0

    lax.fori_loop(0, CHUNK, zero_body, 0)
    _zero_acc_slice(rows.at[0], acc, row0)
    plsc.subcore_barrier()

    table = g_hbm.at[c]
    gsems = [gs0, gs1]
    ssems = [ss0, ss1]
    half = NCHUNK // 2
    for h in range(2):
        base = h * half
        pltpu.sync_copy(srcp_hbm.at[t].at[pl.ds(base, half)], srcv)
        gds = {}
        for b in range(NBUF):
            gds[b] = pltpu.async_copy(table.at[srcv.at[b]], rows.at[b],
                                      gsems[b])
        sds = {}
        for jj in range(half):
            j = base + jj
            b = jj % NBUF
            gds[b].wait()
            sds[b] = pltpu.async_copy(rows.at[b], acc.at[dstv.at[j]],
                                      ssems[b], add=True)
            njj = jj + NBUF
            if njj < half:
                ob = njj % NBUF
                sds.pop(ob).wait()
                gds[ob] = pltpu.async_copy(table.at[srcv.at[njj]],
                                           rows.at[ob], gsems[ob])
        for b in sorted(sds):
            sds[b].wait()

    plsc.subcore_barrier()

    last = NN - (NS - 1) * ROWS_PER_TILE

    @pl.when(t < NS - 1)
    def _():
        pltpu.sync_copy(acc.at[pl.ds(row0, ROWS_PER_TILE)],
                        out_hbm.at[c].at[pl.ds(row0, ROWS_PER_TILE)])

    @pl.when(t == NS - 1)
    def _():
        pltpu.sync_copy(acc.at[pl.ds((NS - 1) * ROWS_PER_TILE, last)],
                        out_hbm.at[c].at[pl.ds((NS - 1) * ROWS_PER_TILE, last)])


@functools.lru_cache(maxsize=1)
def _sc_kernels():
    mesh = plsc.VectorSubcoreMesh(
        core_axis_name="c", subcore_axis_name="s",
        num_cores=NC, num_subcores=NS)
    deg_kernel = pl.kernel(
        _deg_body,
        out_type=jax.ShapeDtypeStruct((NC, ACC_ROWS, HALF), jnp.float32),
        mesh=mesh,
        scratch_types=[
            pltpu.VMEM((NCHUNK, CHUNK), jnp.int32),
            pltpu.VMEM((CHUNK, HALF), jnp.float32),
            pltpu.VMEM_SHARED((ACC_ROWS, HALF), jnp.float32),
            pltpu.SemaphoreType.DMA,
            pltpu.SemaphoreType.DMA,
        ],
    )
    scatter_kernel = pl.kernel(
        _scatter_body,
        out_type=jax.ShapeDtypeStruct((NC, NN, HALF), jnp.float32),
        mesh=mesh,
        scratch_types=[
            pltpu.VMEM((NCHUNK // 2, CHUNK), jnp.int32),
            pltpu.VMEM((NCHUNK, CHUNK), jnp.int32),
            pltpu.VMEM((NBUF, CHUNK, HALF), jnp.float32),
            pltpu.VMEM_SHARED((ACC_ROWS, HALF), jnp.float32),
            pltpu.SemaphoreType.DMA,
            pltpu.SemaphoreType.DMA,
            pltpu.SemaphoreType.DMA,
            pltpu.SemaphoreType.DMA,
        ],
    )
    return deg_kernel, scatter_kernel


def _k_pre(x_ref, w_ref, b_ref, y_ref, s_ref, q_ref):
    r = pl.program_id(0)
    yb = jnp.dot(x_ref[...], w_ref[...], preferred_element_type=jnp.float32)
    yb = yb + b_ref[...]
    y_ref[...] = yb

    @pl.when(r == 0)
    def _():
        s_ref[...] = jnp.zeros_like(s_ref)
        q_ref[...] = jnp.zeros_like(q_ref)

    s_ref[...] += jnp.sum(yb, axis=0, keepdims=True)
    q_ref[...] += jnp.sum(yb * yb, axis=0, keepdims=True)


def _bn_scale_shift(s_ref, q_ref, g_ref, be_ref):
    mu = s_ref[...] / NN
    var = q_ref[...] / NN - mu * mu
    sc = g_ref[...] * lax.rsqrt(var + EPS)
    sh = be_ref[...] - mu * sc
    return sc, sh


def _k_bn_mm(y_ref, s_ref, q_ref, g_ref, be_ref, deg_ref, w_ref,
             h0_ref, gd_ref, dinv_ref):
    sc, sh = _bn_scale_shift(s_ref, q_ref, g_ref, be_ref)
    dinv = lax.rsqrt(deg_ref[...] + 1.0)
    dinv_ref[...] = dinv
    h = jnp.maximum(y_ref[...] * sc + sh, 0.0)
    h0_ref[...] = h
    z = jnp.dot(dinv * h, w_ref[...], preferred_element_type=jnp.float32)
    gd_ref[0] = z[:, :HALF]
    gd_ref[1] = z[:, HALF:]


def _k_conv_mm(sa_ref, gd_ref, dinv_ref, bc_ref, h0_ref, w_ref, gdn_ref):
    u0 = sa_ref[0] + gd_ref[0]
    u1 = sa_ref[1] + gd_ref[1]
    u = jnp.concatenate([u0, u1], axis=1)
    h = dinv_ref[...] * u + bc_ref[...] + h0_ref[...]
    z = jnp.dot(dinv_ref[...] * h, w_ref[...], preferred_element_type=jnp.float32)
    gdn_ref[0] = z[:, :HALF]
    gdn_ref[1] = z[:, HALF:]


def _k_conv_post(sa_ref, gd_ref, dinv_ref, bc_ref, h0_ref, w_ref, b_ref,
                 wout_ref, s_ref, q_ref):
    r = pl.program_id(0)
    u0 = sa_ref[0] + gd_ref[0]
    u1 = sa_ref[1] + gd_ref[1]
    u = jnp.concatenate([u0, u1], axis=1)
    h = dinv_ref[...] * u + bc_ref[...] + h0_ref[...]
    wb = jnp.dot(h, w_ref[...], preferred_element_type=jnp.float32) + b_ref[...]
    wout_ref[...] = wb

    @pl.when(r == 0)
    def _():
        s_ref[...] = jnp.zeros_like(s_ref)
        q_ref[...] = jnp.zeros_like(q_ref)

    s_ref[...] += jnp.sum(wb, axis=0, keepdims=True)
    q_ref[...] += jnp.sum(wb * wb, axis=0, keepdims=True)


def _k_final(w_ref, s_ref, q_ref, g_ref, be_ref, wf_ref, bf_ref, out_ref):
    sc, sh = _bn_scale_shift(s_ref, q_ref, g_ref, be_ref)
    h = jnp.maximum(w_ref[...] * sc + sh, 0.0)
    tt = jnp.dot(h, wf_ref[...], preferred_element_type=jnp.float32) + bf_ref[...]
    m = jnp.max(tt, axis=1, keepdims=True)
    e = jnp.exp(tt - m)
    lse = jnp.log(jnp.sum(e, axis=1, keepdims=True))
    out_ref[...] = tt - m - lse


def _row_spec(cols):
    return pl.BlockSpec((BR, cols), lambda r: (r, 0))


def _full_spec(shape):
    return pl.BlockSpec(shape, lambda r: tuple(0 for _ in shape))


def _half_spec():
    return pl.BlockSpec((NC, BR, HALF), lambda r: (0, r, 0))


_VEC = _row_spec(1)
_STAT = _full_spec((1, HH))


def kernel(x, edge_index, W_pre, b_pre, g_pre, be_pre, Wc1, bc1, Wc2, bc2,
           Wc3, bc3, W_post, b_post, g_post, be_post, W_f, b_f):
    f32 = jnp.float32
    src = edge_index[0]
    dst = edge_index[1]

    pad_s = jnp.zeros((NS, E_PAD - E_PER_TILE), jnp.int32)
    pad_d = jnp.full((NS, E_PAD - E_PER_TILE), NN, jnp.int32)
    srcp = jnp.concatenate([src.reshape(NS, E_PER_TILE), pad_s], axis=1)
    srcp = srcp.reshape(NS, NCHUNK, CHUNK)
    dstp = jnp.concatenate([dst.reshape(NS, E_PER_TILE), pad_d], axis=1)
    dstp = dstp.reshape(NS, NCHUNK, CHUNK)
    _deg_kernel, _scatter_kernel = _sc_kernels()
    hist = _deg_kernel(dstp)
    deg = hist[0, :NN, 0:1] + hist[1, :NN, 0:1]

    b_pre2 = b_pre.reshape(1, HH)
    g_pre2 = g_pre.reshape(1, HH)
    be_pre2 = be_pre.reshape(1, HH)
    bc12 = bc1.reshape(1, HH)
    bc22 = bc2.reshape(1, HH)
    bc32 = bc3.reshape(1, HH)
    b_post2 = b_post.reshape(1, HH)
    g_post2 = g_post.reshape(1, HH)
    be_post2 = be_post.reshape(1, HH)
    b_f2 = b_f.reshape(1, COUT)

    y, s1, q1 = pl.pallas_call(
        _k_pre,
        grid=(ROWB,),
        in_specs=[_row_spec(DIN), _full_spec((DIN, HH)), _STAT],
        out_specs=[_row_spec(HH), _STAT, _STAT],
        out_shape=[
            jax.ShapeDtypeStruct((NN, HH), f32),
            jax.ShapeDtypeStruct((1, HH), f32),
            jax.ShapeDtypeStruct((1, HH), f32),
        ],
    )(x, W_pre, b_pre2)

    h0, gd1, dinv = pl.pallas_call(
        _k_bn_mm,
        grid=(ROWB,),
        in_specs=[_row_spec(HH), _STAT, _STAT, _STAT, _STAT, _VEC,
                  _full_spec((HH, HH))],
        out_specs=[_row_spec(HH), _half_spec(), _VEC],
        out_shape=[
            jax.ShapeDtypeStruct((NN, HH), f32),
            jax.ShapeDtypeStruct((NC, NN, HALF), f32),
            jax.ShapeDtypeStruct((NN, 1), f32),
        ],
    )(y, s1, q1, g_pre2, be_pre2, deg, Wc1)

    sa1 = _scatter_kernel(gd1, srcp, dstp)

    conv_call = pl.pallas_call(
        _k_conv_mm,
        grid=(ROWB,),
        in_specs=[_half_spec(), _half_spec(), _VEC, _STAT, _row_spec(HH),
                  _full_spec((HH, HH))],
        out_specs=_half_spec(),
        out_shape=jax.ShapeDtypeStruct((NC, NN, HALF), f32),
    )

    gd2 = conv_call(sa1, gd1, dinv, bc12, h0, Wc2)
    sa2 = _scatter_kernel(gd2, srcp, dstp)

    gd3 = conv_call(sa2, gd2, dinv, bc22, h0, Wc3)
    sa3 = _scatter_kernel(gd3, srcp, dstp)

    w, s2, q2 = pl.pallas_call(
        _k_conv_post,
        grid=(ROWB,),
        in_specs=[_half_spec(), _half_spec(), _VEC, _STAT, _row_spec(HH),
                  _full_spec((HH, HH)), _STAT],
        out_specs=[_row_spec(HH), _STAT, _STAT],
        out_shape=[
            jax.ShapeDtypeStruct((NN, HH), f32),
            jax.ShapeDtypeStruct((1, HH), f32),
            jax.ShapeDtypeStruct((1, HH), f32),
        ],
    )(sa3, gd3, dinv, bc32, h0, W_post, b_post2)

    out = pl.pallas_call(
        _k_final,
        grid=(ROWB,),
        in_specs=[_row_spec(HH), _STAT, _STAT, _STAT, _STAT,
                  _full_spec((HH, COUT)), _full_spec((1, COUT))],
        out_specs=_row_spec(COUT),
        out_shape=jax.ShapeDtypeStruct((NN, COUT), f32),
    )(w, s2, q2, g_post2, be_post2, W_f, b_f2)

    return out


# speedup vs baseline: 8.7064x; 1.0025x over previous
import functools

import jax
import jax.numpy as jnp
from jax import lax
from jax.experimental import pallas as pl
from jax.experimental.pallas import tpu as pltpu
from jax.experimental.pallas import tpu_sc as plsc

NN = 10000
EE = 160000
DIN = 256
HH = 256
COUT = 64

NC = 2
NS = 16
LL = 16

HALF = HH // 2
E_PER_TILE = EE // NS
CHUNK = 128
NCHUNK = 80
NBUF = 2
E_PAD = NCHUNK * CHUNK
ACC_ROWS = 10112
ROWS_PER_TILE = ACC_ROWS // NS
ROWB = 2
BR = NN // ROWB
EPS = 1e-5

def _zero_acc_slice(zbuf, acc, row0):
    nfull = ROWS_PER_TILE // CHUNK
    rem = ROWS_PER_TILE - nfull * CHUNK
    for k in range(nfull):
        pltpu.sync_copy(zbuf.at[pl.ds(0, CHUNK)],
                        acc.at[pl.ds(row0 + k * CHUNK, CHUNK)])
    if rem:
        pltpu.sync_copy(zbuf.at[pl.ds(0, rem)],
                        acc.at[pl.ds(row0 + nfull * CHUNK, rem)])


def _deg_body(dstp_hbm, out_hbm, dstv, ones_rows, acc, ds0, ds1):
    dsems = [ds0, ds1]
    c = lax.axis_index("c")
    t = lax.axis_index("s")

    row0 = pl.multiple_of(t * ROWS_PER_TILE, 8)
    pltpu.sync_copy(dstp_hbm.at[t], dstv)
    zeros16 = jnp.zeros((LL,), jnp.float32)

    def zero_body(j, _):
        for kk in range(HALF // LL):
            ones_rows[j, pl.ds(kk * LL, LL)] = zeros16
        return 0

    lax.fori_loop(0, CHUNK, zero_body, 0)
    _zero_acc_slice(ones_rows, acc, row0)
    ones16 = jnp.ones((LL,), jnp.float32)

    def ones_body(j, _):
        for kk in range(HALF // LL):
            ones_rows[j, pl.ds(kk * LL, LL)] = ones16
        return 0

    lax.fori_loop(0, CHUNK, ones_body, 0)
    plsc.subcore_barrier()

    half_chunks = NCHUNK // 2

    def scatter_ones(lo, hi):
        pend = {}
        for j in range(lo, hi):
            b = j % 2
            if b in pend:
                pend[b].wait()
            pend[b] = pltpu.async_copy(ones_rows, acc.at[dstv.at[j]],
                                       dsems[b], add=True)
        for b in sorted(pend):
            pend[b].wait()

    @pl.when(c == 0)
    def _():
        scatter_ones(0, half_chunks)

    @pl.when(c == 1)
    def _():
        scatter_ones(half_chunks, NCHUNK)

    plsc.subcore_barrier()
    pltpu.sync_copy(acc.at[pl.ds(row0, ROWS_PER_TILE)],
                    out_hbm.at[c].at[pl.ds(row0, ROWS_PER_TILE)])


def _scatter_body(g_hbm, srcp_hbm, dstp_hbm, out_hbm,
                  srcv, dstv, rows, acc, gs0, gs1, ss0, ss1):
    c = lax.axis_index("c")
    t = lax.axis_index("s")

    row0 = pl.multiple_of(t * ROWS_PER_TILE, 8)
    pltpu.sync_copy(dstp_hbm.at[t], dstv)
    zeros16 = jnp.zeros((LL,), jnp.float32)

    def zero_body(j, _):
        for kk in range(HALF // LL):
            rows[0, j, pl.ds(kk * LL, LL)] = zeros16
        return 0

    lax.fori_loop(0, CHUNK, zero_body, 0)
    _zero_acc_slice(rows.at[0], acc, row0)
    plsc.subcore_barrier()

    table = g_hbm.at[c]
    gsems = [gs0, gs1]
    ssems = [ss0, ss1]
    half = NCHUNK // 2
    for h in range(2):
        base = h * half
        pltpu.sync_copy(srcp_hbm.at[t].at[pl.ds(base, half)], srcv)
        gds = {}
        for b in range(NBUF):
            gds[b] = pltpu.async_copy(table.at[srcv.at[b]], rows.at[b],
                                      gsems[b])
        sds = {}
        for jj in range(half):
            j = base + jj
            b = jj % NBUF
            gds[b].wait()
            sds[b] = pltpu.async_copy(rows.at[b], acc.at[dstv.at[j]],
                                      ssems[b], add=True)
            njj = jj + NBUF
            if njj < half:
                ob = njj % NBUF
                sds.pop(ob).wait()
                gds[ob] = pltpu.async_copy(table.at[srcv.at[njj]],
                                           rows.at[ob], gsems[ob])
        for b in sorted(sds):
            sds[b].wait()

    plsc.subcore_barrier()

    last = NN - (NS - 1) * ROWS_PER_TILE

    @pl.when(t < NS - 1)
    def _():
        pltpu.sync_copy(acc.at[pl.ds(row0, ROWS_PER_TILE)],
                        out_hbm.at[c].at[pl.ds(row0, ROWS_PER_TILE)])

    @pl.when(t == NS - 1)
    def _():
        pltpu.sync_copy(acc.at[pl.ds((NS - 1) * ROWS_PER_TILE, last)],
                        out_hbm.at[c].at[pl.ds((NS - 1) * ROWS_PER_TILE, last)])


@functools.lru_cache(maxsize=1)
def _sc_kernels():
    mesh = plsc.VectorSubcoreMesh(
        core_axis_name="c", subcore_axis_name="s",
        num_cores=NC, num_subcores=NS)
    deg_kernel = pl.kernel(
        _deg_body,
        out_type=jax.ShapeDtypeStruct((NC, ACC_ROWS, HALF), jnp.float32),
        mesh=mesh,
        scratch_types=[
            pltpu.VMEM((NCHUNK, CHUNK), jnp.int32),
            pltpu.VMEM((CHUNK, HALF), jnp.float32),
            pltpu.VMEM_SHARED((ACC_ROWS, HALF), jnp.float32),
            pltpu.SemaphoreType.DMA,
            pltpu.SemaphoreType.DMA,
        ],
    )
    scatter_kernel = pl.kernel(
        _scatter_body,
        out_type=jax.ShapeDtypeStruct((NC, NN, HALF), jnp.float32),
        mesh=mesh,
        scratch_types=[
            pltpu.VMEM((NCHUNK // 2, CHUNK), jnp.int32),
            pltpu.VMEM((NCHUNK, CHUNK), jnp.int32),
            pltpu.VMEM((NBUF, CHUNK, HALF), jnp.float32),
            pltpu.VMEM_SHARED((ACC_ROWS, HALF), jnp.float32),
            pltpu.SemaphoreType.DMA,
            pltpu.SemaphoreType.DMA,
            pltpu.SemaphoreType.DMA,
            pltpu.SemaphoreType.DMA,
        ],
    )
    return deg_kernel, scatter_kernel


def _k_pre(x_ref, w_ref, b_ref, y_ref, s_ref, q_ref):
    r = pl.program_id(0)
    yb = jnp.dot(x_ref[...], w_ref[...], preferred_element_type=jnp.float32)
    yb = yb + b_ref[...]
    y_ref[...] = yb

    @pl.when(r == 0)
    def _():
        s_ref[...] = jnp.zeros_like(s_ref)
        q_ref[...] = jnp.zeros_like(q_ref)

    s_ref[...] += jnp.sum(yb, axis=0, keepdims=True)
    q_ref[...] += jnp.sum(yb * yb, axis=0, keepdims=True)


def _bn_scale_shift(s_ref, q_ref, g_ref, be_ref):
    mu = s_ref[...] / NN
    var = q_ref[...] / NN - mu * mu
    sc = g_ref[...] * lax.rsqrt(var + EPS)
    sh = be_ref[...] - mu * sc
    return sc, sh


def _k_bn_mm(y_ref, s_ref, q_ref, g_ref, be_ref, deg_ref, w_ref,
             h0_ref, gd_ref, dinv_ref):
    sc, sh = _bn_scale_shift(s_ref, q_ref, g_ref, be_ref)
    dinv = lax.rsqrt(deg_ref[...] + 1.0)
    dinv_ref[...] = dinv
    h = jnp.maximum(y_ref[...] * sc + sh, 0.0)
    h0_ref[...] = h
    z = jnp.dot(dinv * h, w_ref[...], preferred_element_type=jnp.float32)
    gd_ref[0] = z[:, :HALF]
    gd_ref[1] = z[:, HALF:]


def _k_conv_mm(sa_ref, gd_ref, dinv_ref, bc_ref, h0_ref, w_ref, gdn_ref):
    u0 = sa_ref[0] + gd_ref[0]
    u1 = sa_ref[1] + gd_ref[1]
    u = jnp.concatenate([u0, u1], axis=1)
    h = dinv_ref[...] * u + bc_ref[...] + h0_ref[...]
    z = jnp.dot(dinv_ref[...] * h, w_ref[...], preferred_element_type=jnp.float32)
    gdn_ref[0] = z[:, :HALF]
    gdn_ref[1] = z[:, HALF:]


def _k_conv_post(sa_ref, gd_ref, dinv_ref, bc_ref, h0_ref, w_ref, b_ref,
                 wout_ref, s_ref, q_ref):
    r = pl.program_id(0)
    u0 = sa_ref[0] + gd_ref[0]
    u1 = sa_ref[1] + gd_ref[1]
    u = jnp.concatenate([u0, u1], axis=1)
    h = dinv_ref[...] * u + bc_ref[...] + h0_ref[...]
    wb = jnp.dot(h, w_ref[...], preferred_element_type=jnp.float32) + b_ref[...]
    wout_ref[...] = wb

    @pl.when(r == 0)
    def _():
        s_ref[...] = jnp.zeros_like(s_ref)
        q_ref[...] = jnp.zeros_like(q_ref)

    s_ref[...] += jnp.sum(wb, axis=0, keepdims=True)
    q_ref[...] += jnp.sum(wb * wb, axis=0, keepdims=True)


def _k_final(w_ref, s_ref, q_ref, g_ref, be_ref, wf_ref, bf_ref, out_ref):
    sc, sh = _bn_scale_shift(s_ref, q_ref, g_ref, be_ref)
    h = jnp.maximum(w_ref[...] * sc + sh, 0.0)
    tt = jnp.dot(h, wf_ref[...], preferred_element_type=jnp.float32) + bf_ref[...]
    m = jnp.max(tt, axis=1, keepdims=True)
    e = jnp.exp(tt - m)
    lse = jnp.log(jnp.sum(e, axis=1, keepdims=True))
    out_ref[...] = tt - m - lse


def _row_spec(cols):
    return pl.BlockSpec((BR, cols), lambda r: (r, 0))


def _full_spec(shape):
    return pl.BlockSpec(shape, lambda r: tuple(0 for _ in shape))


def _half_spec():
    return pl.BlockSpec((NC, BR, HALF), lambda r: (0, r, 0))


_VEC = _row_spec(1)
_STAT = _full_spec((1, HH))


def kernel(x, edge_index, W_pre, b_pre, g_pre, be_pre, Wc1, bc1, Wc2, bc2,
           Wc3, bc3, W_post, b_post, g_post, be_post, W_f, b_f):
    f32 = jnp.float32
    src = edge_index[0]
    dst = edge_index[1]

    pad_s = jnp.zeros((NS, E_PAD - E_PER_TILE), jnp.int32)
    pad_d = jnp.full((NS, E_PAD - E_PER_TILE), NN, jnp.int32)
    srcp = jnp.concatenate([src.reshape(NS, E_PER_TILE), pad_s], axis=1)
    srcp = srcp.reshape(NS, NCHUNK, CHUNK)
    dstp = jnp.concatenate([dst.reshape(NS, E_PER_TILE), pad_d], axis=1)
    dstp = dstp.reshape(NS, NCHUNK, CHUNK)
    _deg_kernel, _scatter_kernel = _sc_kernels()
    hist = _deg_kernel(dstp)
    deg = hist[0, :NN, 0:1] + hist[1, :NN, 0:1]

    b_pre2 = b_pre.reshape(1, HH)
    g_pre2 = g_pre.reshape(1, HH)
    be_pre2 = be_pre.reshape(1, HH)
    bc12 = bc1.reshape(1, HH)
    bc22 = bc2.reshape(1, HH)
    bc32 = bc3.reshape(1, HH)
    b_post2 = b_post.reshape(1, HH)
    g_post2 = g_post.reshape(1, HH)
    be_post2 = be_post.reshape(1, HH)
    b_f2 = b_f.reshape(1, COUT)

    y, s1, q1 = pl.pallas_call(
        _k_pre,
        grid=(ROWB,),
        in_specs=[_row_spec(DIN), _full_spec((DIN, HH)), _STAT],
        out_specs=[_row_spec(HH), _STAT, _STAT],
        out_shape=[
            jax.ShapeDtypeStruct((NN, HH), f32),
            jax.ShapeDtypeStruct((1, HH), f32),
            jax.ShapeDtypeStruct((1, HH), f32),
        ],
    )(x, W_pre, b_pre2)

    h0, gd1, dinv = pl.pallas_call(
        _k_bn_mm,
        grid=(ROWB,),
        in_specs=[_row_spec(HH), _STAT, _STAT, _STAT, _STAT, _VEC,
                  _full_spec((HH, HH))],
        out_specs=[_row_spec(HH), _half_spec(), _VEC],
        out_shape=[
            jax.ShapeDtypeStruct((NN, HH), f32),
            jax.ShapeDtypeStruct((NC, NN, HALF), f32),
            jax.ShapeDtypeStruct((NN, 1), f32),
        ],
    )(y, s1, q1, g_pre2, be_pre2, deg, Wc1)

    sa1 = _scatter_kernel(gd1, srcp, dstp)

    conv_call = pl.pallas_call(
        _k_conv_mm,
        grid=(ROWB,),
        in_specs=[_half_spec(), _half_spec(), _VEC, _STAT, _row_spec(HH),
                  _full_spec((HH, HH))],
        out_specs=_half_spec(),
        out_shape=jax.ShapeDtypeStruct((NC, NN, HALF), f32),
    )

    gd2 = conv_call(sa1, gd1, dinv, bc12, h0, Wc2)
    sa2 = _scatter_kernel(gd2, srcp, dstp)

    gd3 = conv_call(sa2, gd2, dinv, bc22, h0, Wc3)
    sa3 = _scatter_kernel(gd3, srcp, dstp)

    w, s2, q2 = pl.pallas_call(
        _k_conv_post,
        grid=(ROWB,),
        in_specs=[_half_spec(), _half_spec(), _VEC, _STAT, _row_spec(HH),
                  _full_spec((HH, HH)), _STAT],
        out_specs=[_row_spec(HH), _STAT, _STAT],
        out_shape=[
            jax.ShapeDtypeStruct((NN, HH), f32),
            jax.ShapeDtypeStruct((1, HH), f32),
            jax.ShapeDtypeStruct((1, HH), f32),
        ],
    )(sa3, gd3, dinv, bc32, h0, W_post, b_post2)

    out = pl.pallas_call(
        _k_final,
        grid=(ROWB,),
        in_specs=[_row_spec(HH), _STAT, _STAT, _STAT, _STAT,
                  _full_spec((HH, COUT)), _full_spec((1, COUT))],
        out_specs=_row_spec(COUT),
        out_shape=jax.ShapeDtypeStruct((NN, COUT), f32),
    )(w, s2, q2, g_post2, be_post2, W_f, b_f2)

    return out
